# Initial kernel scaffold; baseline (speedup 1.0000x reference)
import jax
import jax.numpy as jnp
from jax.experimental import pallas as pl


def kernel(x, emb_table, W, b):
    raise NotImplementedError("write your pallas kernel here")


# speedup vs baseline: 1.4455x; 1.4455x over previous
import functools

import jax
import jax.numpy as jnp
from jax import lax
from jax.experimental import pallas as pl
from jax.experimental.pallas import tpu as pltpu
from jax.experimental.pallas import tpu_sc as plsc

_B = 16384
_D = 64
_O = 128
_NC = 2
_NS = 16
_NW = _NC * _NS
_BPW = _B // _NW
_CH = 128
_NCH = _BPW // _CH


def _sc_gather(table, idx1, idx2):
  mesh = plsc.VectorSubcoreMesh(core_axis_name="c", subcore_axis_name="s")

  @functools.partial(
      pl.kernel,
      mesh=mesh,
      out_type=[
          jax.ShapeDtypeStruct((_B, _D), jnp.float32),
          jax.ShapeDtypeStruct((_B, _D), jnp.float32),
      ],
      scratch_types=[
          pltpu.VMEM((_NCH, _CH), jnp.int32),
          pltpu.VMEM((_NCH, _CH), jnp.int32),
          pltpu.VMEM((_BPW, _D), jnp.float32),
          pltpu.VMEM((_BPW, _D), jnp.float32),
          pltpu.SemaphoreType.DMA,
      ],
      compiler_params=pltpu.CompilerParams(use_tc_tiling_on_sc=False),
  )
  def gather_kernel(table_hbm, idx1_hbm, idx2_hbm, e1_hbm, e2_hbm,
                    idx1_v, idx2_v, rows1_v, rows2_v, sem):
    wid = lax.axis_index("s") * _NC + lax.axis_index("c")
    base = wid * _BPW
    row0 = wid * _NCH
    pltpu.sync_copy(idx1_hbm.at[pl.ds(row0, _NCH)], idx1_v)
    pltpu.sync_copy(idx2_hbm.at[pl.ds(row0, _NCH)], idx2_v)
    copies = []
    for j in range(_NCH):
      copies.append(pltpu.async_copy(
          table_hbm.at[idx1_v.at[j]], rows1_v.at[pl.ds(j * _CH, _CH)], sem))
      copies.append(pltpu.async_copy(
          table_hbm.at[idx2_v.at[j]], rows2_v.at[pl.ds(j * _CH, _CH)], sem))
    for c in copies:
      c.wait()
    pltpu.sync_copy(rows1_v, e1_hbm.at[pl.ds(base, _BPW)])
    pltpu.sync_copy(rows2_v, e2_hbm.at[pl.ds(base, _BPW)])

  return gather_kernel(table,
                       idx1.reshape(_NW * _NCH, _CH),
                       idx2.reshape(_NW * _NCH, _CH))


_BM = 1024


def _tc_project(e1, e2, W, b2d):
  def mm_kernel(e1_ref, e2_ref, w_ref, b_ref, o_ref):
    acc = lax.dot_general(e1_ref[...], w_ref[:, :_D],
                          (((1,), (1,)), ((), ())),
                          preferred_element_type=jnp.float32)
    acc += lax.dot_general(e2_ref[...], w_ref[:, _D:],
                           (((1,), (1,)), ((), ())),
                           preferred_element_type=jnp.float32)
    o_ref[...] = acc + b_ref[...]

  return pl.pallas_call(
      mm_kernel,
      grid=(_B // _BM,),
      in_specs=[
          pl.BlockSpec((_BM, _D), lambda i: (i, 0)),
          pl.BlockSpec((_BM, _D), lambda i: (i, 0)),
          pl.BlockSpec((_O, 2 * _D), lambda i: (0, 0)),
          pl.BlockSpec((1, _O), lambda i: (0, 0)),
      ],
      out_specs=pl.BlockSpec((_BM, _O), lambda i: (i, 0)),
      out_shape=jax.ShapeDtypeStruct((_B, _O), jnp.float32),
  )(e1, e2, W, b2d)


def kernel(x, emb_table, W, b):
  idx1 = x[:, 0].astype(jnp.int32)
  idx2 = x[:, 1].astype(jnp.int32)
  e1, e2 = _sc_gather(emb_table, idx1, idx2)
  return _tc_project(e1, e2, W, b.reshape(1, _O))


# speedup vs baseline: 1.5958x; 1.1040x over previous
import functools

import jax
import jax.numpy as jnp
from jax import lax
from jax.experimental import pallas as pl
from jax.experimental.pallas import tpu as pltpu
from jax.experimental.pallas import tpu_sc as plsc

_B = 16384
_D = 64
_O = 128
_G = 2 * _B
_NC = 2
_NS = 16
_NW = _NC * _NS
_RPW = _G // _NW
_CH = 128
_NCH = _RPW // _CH


def _sc_gather(table, idx):
  mesh = plsc.VectorSubcoreMesh(core_axis_name="c", subcore_axis_name="s")

  @functools.partial(
      pl.kernel,
      mesh=mesh,
      out_type=jax.ShapeDtypeStruct((_G, _D), jnp.float32),
      scratch_types=[
          pltpu.VMEM((_NCH, _CH), jnp.int32),
          pltpu.VMEM((_RPW, _D), jnp.float32),
          pltpu.SemaphoreType.DMA,
      ],
      compiler_params=pltpu.CompilerParams(use_tc_tiling_on_sc=False),
  )
  def gather_kernel(table_hbm, idx_hbm, out_hbm, idx_v, rows_v, sem):
    wid = lax.axis_index("s") * _NC + lax.axis_index("c")
    pltpu.sync_copy(idx_hbm.at[pl.ds(wid * _NCH, _NCH)], idx_v)
    copies = []
    for j in range(_NCH):
      copies.append(pltpu.async_copy(
          table_hbm.at[idx_v.at[j]], rows_v.at[pl.ds(j * _CH, _CH)], sem))
    for c in copies:
      c.wait()
    pltpu.sync_copy(rows_v, out_hbm.at[pl.ds(wid * _RPW, _RPW)])

  return gather_kernel(table, idx.reshape(_NW * _NCH, _CH))


_BM = 1024


def _tc_project(cat, W, b2d):
  def mm_kernel(cat_ref, w_ref, b_ref, o_ref):
    o_ref[...] = lax.dot_general(
        cat_ref[...], w_ref[...], (((1,), (1,)), ((), ())),
        preferred_element_type=jnp.float32) + b_ref[...]

  return pl.pallas_call(
      mm_kernel,
      grid=(_B // _BM,),
      in_specs=[
          pl.BlockSpec((_BM, 2 * _D), lambda i: (i, 0)),
          pl.BlockSpec((_O, 2 * _D), lambda i: (0, 0)),
          pl.BlockSpec((1, _O), lambda i: (0, 0)),
      ],
      out_specs=pl.BlockSpec((_BM, _O), lambda i: (i, 0)),
      out_shape=jax.ShapeDtypeStruct((_B, _O), jnp.float32),
  )(cat, W, b2d)


def kernel(x, emb_table, W, b):
  idx = x.astype(jnp.int32)
  rows = _sc_gather(emb_table, idx)
  cat = rows.reshape(_B, 2 * _D)
  return _tc_project(cat, W, b.reshape(1, _O))
